# baseline (speedup 1.0000x reference)
import jax
import jax.numpy as jnp
import numpy as np
from jax.experimental import pallas as pl
from jax.experimental.pallas import tpu as pltpu

DIM = 384
HEADS = 12
HD = DIM // HEADS
WS = 7
H = 28
W = 28
B = 4
NW = 16
N = WS * WS
NP = 56
HIDDEN = 1536
TOK = B * H * W
TOKP = B * NW * NP
SCALE = HD ** -0.5


def _rel_pos_index(ws):
    coords = np.stack(np.meshgrid(np.arange(ws), np.arange(ws), indexing='ij'))
    cf = coords.reshape(2, -1)
    rel = cf[:, :, None] - cf[:, None, :]
    rel = rel.transpose(1, 2, 0).copy()
    rel[:, :, 0] += ws - 1
    rel[:, :, 1] += ws - 1
    rel[:, :, 0] *= 2 * ws - 1
    return rel.sum(-1)

_RPI_FLAT = np.asarray(_rel_pos_index(WS).reshape(-1), dtype=np.int32)


def _body(x_ref, g1_ref, b1_ref, wqkv_ref, bqkv_ref, bias_ref,
          wp_ref, bp_ref, g2_ref, be2_ref, w1_ref, bm1_ref, w2_ref, bm2_ref,
          o_ref, qkv_s, att_s, xwb, awb, qbuf):
    f32 = jnp.float32
    pid = pl.program_id(0)

    @pl.when(pid < 8)
    def _phase_qkv():
        for wl in range(2):
            for wc in range(4):
                for i in range(WS):
                    dst = (wl * 4 + wc) * NP + i * WS
                    src = wl * 196 + i * 28 + wc * WS
                    xwb[dst:dst + WS, :] = x_ref[src:src + WS, :]
                xwb[(wl * 4 + wc) * NP + N:(wl * 4 + wc) * NP + NP, :] = (
                    jnp.zeros((NP - N, DIM), f32))
        xv = xwb[...]
        mu = jnp.mean(xv, axis=1, keepdims=True)
        var = jnp.mean((xv - mu) ** 2, axis=1, keepdims=True)
        ln = (xv - mu) * jax.lax.rsqrt(var + 1e-5) * g1_ref[...] + b1_ref[...]

        @pl.when(pid < 2)
        def _full_qkv():
            res = jax.lax.dot_general(
                ln, wqkv_ref[...], (((1,), (1,)), ((), ())),
                preferred_element_type=f32) + bqkv_ref[...]
            qkv_s[0, pl.ds(pid * 448, 448), :] = res[:, :DIM] * SCALE
            qkv_s[1, pl.ds(pid * 448, 448), :] = res[:, DIM:2 * DIM]
            qkv_s[2, pl.ds(pid * 448, 448), :] = res[:, 2 * DIM:]

        @pl.when(pid >= 2)
        def _q_only():
            resq = jax.lax.dot_general(
                ln, wqkv_ref[0:DIM, :], (((1,), (1,)), ((), ())),
                preferred_element_type=f32) + bqkv_ref[:, :DIM]
            qkv_s[0, pl.ds(pid * 448, 448), :] = resq * SCALE

    @pl.when((pid >= 8) & (pid < 12))
    def _phase_attn():
        r = pid - 8
        for t in range(4):
            for a in range(4):
                dst = (t * 4 + a) * NP
                qbuf[dst:dst + NP, :] = qkv_s[0, pl.ds((t * 16 + a * 4 + r) * NP, NP), :]
        qf = qbuf[...]
        lanes2 = jax.lax.broadcasted_iota(jnp.int32, (4 * NP, 2 * NP), 1)
        laneso = jax.lax.broadcasted_iota(jnp.int32, (4 * NP, 2 * HD), 1)
        rows16 = jax.lax.broadcasted_iota(jnp.int32, (16 * NP, NP), 0)
        lanes1 = jax.lax.broadcasted_iota(jnp.int32, (16 * NP, NP), 1)
        onehot = (jax.lax.rem(rows16, NP) == lanes1).astype(f32)
        z56 = jnp.zeros((NP, HD), f32)
        o1 = jnp.ones((NP, 1), f32)
        zz1 = jnp.zeros((NP, 1), f32)
        sumcols = jnp.concatenate([
            jnp.concatenate([o1, zz1], axis=1),
            jnp.concatenate([zz1, o1], axis=1)], axis=0)
        qa = [jnp.concatenate([qf[:, i * 2 * HD:(i + 1) * 2 * HD], onehot],
                              axis=1) for i in range(HEADS // 2)]
        for b in range(4):
            kf = qkv_s[1, pl.ds((4 * r + b) * NP, NP), :]
            vf = qkv_s[2, pl.ds((4 * r + b) * NP, NP), :]
            for i in range(HEADS // 2):
                sl = slice(i * 2 * HD, (i + 1) * 2 * HD)
                k1 = kf[:, i * 2 * HD:i * 2 * HD + HD]
                k2 = kf[:, i * 2 * HD + HD:(i + 1) * 2 * HD]
                bd_k = jnp.concatenate([
                    jnp.concatenate([k1, z56], axis=1),
                    jnp.concatenate([z56, k2], axis=1)], axis=0)
                bd_kb = jnp.concatenate([bd_k, bias_ref[i]], axis=1)
                s = jax.lax.dot_general(
                    qa[i], bd_kb, (((1,), (1,)), ((), ())),
                    preferred_element_type=f32)
                v1 = vf[:, i * 2 * HD:i * 2 * HD + HD]
                v2 = vf[:, i * 2 * HD + HD:(i + 1) * 2 * HD]
                bd_v = jnp.concatenate([
                    jnp.concatenate([v1, z56, sumcols[:NP]], axis=1),
                    jnp.concatenate([z56, v2, sumcols[NP:]], axis=1)],
                    axis=0)
                o4 = jnp.zeros((4 * NP, 2 * HD), f32)
                for t in range(4):
                    st = s[t * 4 * NP:(t + 1) * 4 * NP]
                    m1 = jnp.max(st[:, :NP], axis=-1, keepdims=True)
                    m2 = jnp.max(st[:, NP:], axis=-1, keepdims=True)
                    e = jnp.exp(st - jnp.where(lanes2 < NP, m1, m2))
                    oa = jax.lax.dot_general(
                        e, bd_v, (((1,), (0,)), ((), ())),
                        preferred_element_type=f32)
                    rs1 = 1.0 / oa[:, 2 * HD:2 * HD + 1]
                    rs2 = 1.0 / oa[:, 2 * HD + 1:2 * HD + 2]
                    o4 = o4 + oa[:, :2 * HD] * jnp.where(laneso < HD, rs1, rs2)
                o4 = (o4 * 0.25).reshape(4, NP, 2 * HD)
                for a in range(4):
                    att_s[pl.ds((a * 16 + 4 * r + b) * NP, NP), sl] = o4[a]

    @pl.when(pid >= 12)
    def _phase_mlp():
        j = pid - 12
        xwb[...] = att_s[pl.ds(j * 448, 448), :]
        for wl in range(2):
            for wc in range(4):
                for i in range(WS):
                    src = (wl * 4 + wc) * NP + i * WS
                    dst = wl * 196 + i * 28 + wc * WS
                    awb[dst:dst + WS, :] = xwb[src:src + WS, :]
        z = jax.lax.dot_general(
            awb[...], wp_ref[...], (((1,), (1,)), ((), ())),
            preferred_element_type=f32) + bp_ref[...]
        x2 = x_ref[...] + z
        mu = jnp.mean(x2, axis=1, keepdims=True)
        var = jnp.mean((x2 - mu) ** 2, axis=1, keepdims=True)
        ln = (x2 - mu) * jax.lax.rsqrt(var + 1e-5) * g2_ref[...] + be2_ref[...]
        h1 = jax.lax.dot_general(
            ln, w1_ref[...], (((1,), (1,)), ((), ())),
            preferred_element_type=f32) + bm1_ref[...]
        h1 = 0.5 * h1 * (1.0 + jax.lax.erf(h1 * (2.0 ** -0.5)))
        y = jax.lax.dot_general(
            h1, w2_ref[...], (((1,), (1,)), ((), ())),
            preferred_element_type=f32) + bm2_ref[...]
        o_ref[...] = x2 + y


def kernel(x, n1g, n1b, Wqkv, bqkv, rpb, Wproj, bproj, n2g, n2b, W1, b1, W2, b2):
    f32 = jnp.float32
    xf = x.reshape(TOK, DIM)

    rpbg = rpb[_RPI_FLAT].reshape(N, N, HEADS)
    full = jnp.full((HEADS, NP, NP), -1e30, f32).at[:, :N, :N].set(
        rpbg.transpose(2, 0, 1))
    bias2 = full.transpose(0, 2, 1).reshape(HEADS // 2, 2 * NP, NP)

    c0 = lambda i: (0, 0)
    out = pl.pallas_call(
        _body,
        grid=(20,),
        in_specs=[
            pl.BlockSpec((392, DIM),
                         lambda g: (jnp.where(g < 8, g,
                                              jnp.clip(g - 12, 0, 7)), 0)),
            pl.BlockSpec((1, DIM), c0),
            pl.BlockSpec((1, DIM), c0),
            pl.BlockSpec((3 * DIM, DIM), c0),
            pl.BlockSpec((1, 3 * DIM), c0),
            pl.BlockSpec((HEADS // 2, 2 * NP, NP), lambda g: (0, 0, 0)),
            pl.BlockSpec((DIM, DIM), c0),
            pl.BlockSpec((1, DIM), c0),
            pl.BlockSpec((1, DIM), c0),
            pl.BlockSpec((1, DIM), c0),
            pl.BlockSpec((HIDDEN, DIM), c0),
            pl.BlockSpec((1, HIDDEN), c0),
            pl.BlockSpec((DIM, HIDDEN), c0),
            pl.BlockSpec((1, DIM), c0),
        ],
        out_specs=pl.BlockSpec((392, DIM),
                               lambda g: (jnp.where(g >= 12, g - 12, 0), 0)),
        out_shape=jax.ShapeDtypeStruct((TOK, DIM), f32),
        scratch_shapes=[
            pltpu.VMEM((3, TOKP, DIM), f32),
            pltpu.VMEM((TOKP, DIM), f32),
            pltpu.VMEM((448, DIM), f32),
            pltpu.VMEM((392, DIM), f32),
            pltpu.VMEM((16 * NP, DIM), f32),
        ],
    )(xf, n1g.reshape(1, DIM), n1b.reshape(1, DIM), Wqkv,
      bqkv.reshape(1, 3 * DIM), bias2, Wproj, bproj.reshape(1, DIM),
      n2g.reshape(1, DIM), n2b.reshape(1, DIM),
      W1, b1.reshape(1, HIDDEN), W2, b2.reshape(1, DIM))

    return out.reshape(B, H * W, DIM)


# speedup vs baseline: 1.0982x; 1.0982x over previous
import jax
import jax.numpy as jnp
import numpy as np
from jax.experimental import pallas as pl
from jax.experimental.pallas import tpu as pltpu

DIM = 384
HEADS = 12
HD = DIM // HEADS
WS = 7
H = 28
W = 28
B = 4
NW = 16
N = WS * WS
NP = 56
HIDDEN = 1536
TOK = B * H * W
TOKP = B * NW * NP
SCALE = HD ** -0.5


def _rel_pos_index(ws):
    coords = np.stack(np.meshgrid(np.arange(ws), np.arange(ws), indexing='ij'))
    cf = coords.reshape(2, -1)
    rel = cf[:, :, None] - cf[:, None, :]
    rel = rel.transpose(1, 2, 0).copy()
    rel[:, :, 0] += ws - 1
    rel[:, :, 1] += ws - 1
    rel[:, :, 0] *= 2 * ws - 1
    return rel.sum(-1)

_RPI_FLAT = np.asarray(_rel_pos_index(WS).reshape(-1), dtype=np.int32)


def _body(x_ref, g1_ref, b1_ref, wqkv_ref, bqkv_ref, bias_ref,
          wp_ref, bp_ref, g2_ref, be2_ref, w1_ref, bm1_ref, w2_ref, bm2_ref,
          o_ref, qkv_s, att_s, xwb, awb, qbuf):
    f32 = jnp.float32
    pid = pl.program_id(0)

    @pl.when(pid < 8)
    def _phase_qkv():
        for wl in range(2):
            for wc in range(4):
                for i in range(WS):
                    dst = (wl * 4 + wc) * NP + i * WS
                    src = wl * 196 + i * 28 + wc * WS
                    xwb[dst:dst + WS, :] = x_ref[src:src + WS, :]
                xwb[(wl * 4 + wc) * NP + N:(wl * 4 + wc) * NP + NP, :] = (
                    jnp.zeros((NP - N, DIM), f32))
        xv = xwb[...]
        mu = jnp.mean(xv, axis=1, keepdims=True)
        var = jnp.mean((xv - mu) ** 2, axis=1, keepdims=True)
        ln = (xv - mu) * jax.lax.rsqrt(var + 1e-5) * g1_ref[...] + b1_ref[...]

        @pl.when(pid < 2)
        def _full_qkv():
            res = jax.lax.dot_general(
                ln, wqkv_ref[...], (((1,), (1,)), ((), ())),
                preferred_element_type=f32) + bqkv_ref[...]
            qkv_s[0, pl.ds(pid * 448, 448), :] = res[:, :DIM] * SCALE
            qkv_s[1, pl.ds(pid * 448, 448), :] = res[:, DIM:2 * DIM]
            qkv_s[2, pl.ds(pid * 448, 448), :] = res[:, 2 * DIM:]

        @pl.when(pid >= 2)
        def _q_only():
            resq = jax.lax.dot_general(
                ln, wqkv_ref[0:DIM, :], (((1,), (1,)), ((), ())),
                preferred_element_type=f32) + bqkv_ref[:, :DIM]
            qkv_s[0, pl.ds(pid * 448, 448), :] = resq * SCALE

    @pl.when((pid >= 8) & (pid < 12))
    def _phase_attn():
        r = pid - 8
        for t in range(4):
            for a in range(4):
                dst = (t * 4 + a) * NP
                qbuf[dst:dst + NP, :] = qkv_s[0, pl.ds((t * 16 + a * 4 + r) * NP, NP), :]
        qf = qbuf[...]
        lanes2 = jax.lax.broadcasted_iota(jnp.int32, (4 * NP, 2 * NP), 1)
        laneso = jax.lax.broadcasted_iota(jnp.int32, (4 * NP, 2 * HD), 1)
        z56 = jnp.zeros((NP, HD), f32)
        o1 = jnp.ones((NP, 1), f32)
        zz1 = jnp.zeros((NP, 1), f32)
        sumcols = jnp.concatenate([
            jnp.concatenate([o1, zz1], axis=1),
            jnp.concatenate([zz1, o1], axis=1)], axis=0)
        for b in range(4):
            kf = qkv_s[1, pl.ds((4 * r + b) * NP, NP), :]
            vf = qkv_s[2, pl.ds((4 * r + b) * NP, NP), :]
            for i in range(HEADS // 2):
                sl = slice(i * 2 * HD, (i + 1) * 2 * HD)
                k1 = kf[:, i * 2 * HD:i * 2 * HD + HD]
                k2 = kf[:, i * 2 * HD + HD:(i + 1) * 2 * HD]
                bd_k = jnp.concatenate([
                    jnp.concatenate([k1, z56], axis=1),
                    jnp.concatenate([z56, k2], axis=1)], axis=0)
                s = jax.lax.dot_general(
                    qf[:, sl], bd_k, (((1,), (1,)), ((), ())),
                    preferred_element_type=f32)
                s = (s.reshape(16, NP, 2 * NP) + bias_ref[i][None]
                     ).reshape(16 * NP, 2 * NP)
                v1 = vf[:, i * 2 * HD:i * 2 * HD + HD]
                v2 = vf[:, i * 2 * HD + HD:(i + 1) * 2 * HD]
                bd_v = jnp.concatenate([
                    jnp.concatenate([v1, z56, sumcols[:NP]], axis=1),
                    jnp.concatenate([z56, v2, sumcols[NP:]], axis=1)],
                    axis=0)
                o4 = jnp.zeros((4 * NP, 2 * HD), f32)
                for t in range(4):
                    st = s[t * 4 * NP:(t + 1) * 4 * NP]
                    m1 = jnp.max(st[:, :NP], axis=-1, keepdims=True)
                    m2 = jnp.max(st[:, NP:], axis=-1, keepdims=True)
                    e = jnp.exp(st - jnp.where(lanes2 < NP, m1, m2))
                    oa = jax.lax.dot_general(
                        e, bd_v, (((1,), (0,)), ((), ())),
                        preferred_element_type=f32)
                    rs1 = 1.0 / oa[:, 2 * HD:2 * HD + 1]
                    rs2 = 1.0 / oa[:, 2 * HD + 1:2 * HD + 2]
                    o4 = o4 + oa[:, :2 * HD] * jnp.where(laneso < HD, rs1, rs2)
                o4 = (o4 * 0.25).reshape(4, NP, 2 * HD)
                for a in range(4):
                    att_s[pl.ds((a * 16 + 4 * r + b) * NP, NP), sl] = o4[a]

    @pl.when(pid >= 12)
    def _phase_mlp():
        j = pid - 12
        xwb[...] = att_s[pl.ds(j * 448, 448), :]
        for wl in range(2):
            for wc in range(4):
                for i in range(WS):
                    src = (wl * 4 + wc) * NP + i * WS
                    dst = wl * 196 + i * 28 + wc * WS
                    awb[dst:dst + WS, :] = xwb[src:src + WS, :]
        z = jax.lax.dot_general(
            awb[...], wp_ref[...], (((1,), (1,)), ((), ())),
            preferred_element_type=f32) + bp_ref[...]
        x2 = x_ref[...] + z
        mu = jnp.mean(x2, axis=1, keepdims=True)
        var = jnp.mean((x2 - mu) ** 2, axis=1, keepdims=True)
        ln = (x2 - mu) * jax.lax.rsqrt(var + 1e-5) * g2_ref[...] + be2_ref[...]
        h1 = jax.lax.dot_general(
            ln, w1_ref[...], (((1,), (1,)), ((), ())),
            preferred_element_type=f32) + bm1_ref[...]
        h1 = 0.5 * h1 * (1.0 + jax.lax.erf(h1 * (2.0 ** -0.5)))
        y = jax.lax.dot_general(
            h1, w2_ref[...], (((1,), (1,)), ((), ())),
            preferred_element_type=f32) + bm2_ref[...]
        o_ref[...] = x2 + y


def kernel(x, n1g, n1b, Wqkv, bqkv, rpb, Wproj, bproj, n2g, n2b, W1, b1, W2, b2):
    f32 = jnp.float32
    xf = x.reshape(TOK, DIM)

    rpbg = rpb[_RPI_FLAT].reshape(N, N, HEADS)
    full = jnp.full((HEADS, NP, NP), -1e30, f32).at[:, :N, :N].set(
        rpbg.transpose(2, 0, 1))
    bias2 = jnp.concatenate([full[0::2], full[1::2]], axis=2)

    c0 = lambda i: (0, 0)
    out = pl.pallas_call(
        _body,
        grid=(20,),
        in_specs=[
            pl.BlockSpec((392, DIM),
                         lambda g: (jnp.where(g < 8, g,
                                              jnp.clip(g - 12, 0, 7)), 0)),
            pl.BlockSpec((1, DIM), c0),
            pl.BlockSpec((1, DIM), c0),
            pl.BlockSpec((3 * DIM, DIM), c0),
            pl.BlockSpec((1, 3 * DIM), c0),
            pl.BlockSpec((HEADS // 2, NP, 2 * NP), lambda g: (0, 0, 0)),
            pl.BlockSpec((DIM, DIM), c0),
            pl.BlockSpec((1, DIM), c0),
            pl.BlockSpec((1, DIM), c0),
            pl.BlockSpec((1, DIM), c0),
            pl.BlockSpec((HIDDEN, DIM), c0),
            pl.BlockSpec((1, HIDDEN), c0),
            pl.BlockSpec((DIM, HIDDEN), c0),
            pl.BlockSpec((1, DIM), c0),
        ],
        out_specs=pl.BlockSpec((392, DIM),
                               lambda g: (jnp.where(g >= 12, g - 12, 0), 0)),
        out_shape=jax.ShapeDtypeStruct((TOK, DIM), f32),
        scratch_shapes=[
            pltpu.VMEM((3, TOKP, DIM), f32),
            pltpu.VMEM((TOKP, DIM), f32),
            pltpu.VMEM((448, DIM), f32),
            pltpu.VMEM((392, DIM), f32),
            pltpu.VMEM((16 * NP, DIM), f32),
        ],
    )(xf, n1g.reshape(1, DIM), n1b.reshape(1, DIM), Wqkv,
      bqkv.reshape(1, 3 * DIM), bias2, Wproj, bproj.reshape(1, DIM),
      n2g.reshape(1, DIM), n2b.reshape(1, DIM),
      W1, b1.reshape(1, HIDDEN), W2, b2.reshape(1, DIM))

    return out.reshape(B, H * W, DIM)


# speedup vs baseline: 1.1044x; 1.0056x over previous
import jax
import jax.numpy as jnp
import numpy as np
from jax.experimental import pallas as pl
from jax.experimental.pallas import tpu as pltpu

DIM = 384
HEADS = 12
HD = DIM // HEADS
WS = 7
H = 28
W = 28
B = 4
NW = 16
N = WS * WS
NP = 56
HIDDEN = 1536
TOK = B * H * W
TOKP = B * NW * NP
SCALE = HD ** -0.5


def _rel_pos_index(ws):
    coords = np.stack(np.meshgrid(np.arange(ws), np.arange(ws), indexing='ij'))
    cf = coords.reshape(2, -1)
    rel = cf[:, :, None] - cf[:, None, :]
    rel = rel.transpose(1, 2, 0).copy()
    rel[:, :, 0] += ws - 1
    rel[:, :, 1] += ws - 1
    rel[:, :, 0] *= 2 * ws - 1
    return rel.sum(-1)

_RPI_FLAT = np.asarray(_rel_pos_index(WS).reshape(-1), dtype=np.int32)


def _body(x_ref, g1_ref, b1_ref, wqkv_ref, bqkv_ref, bias_ref,
          wp_ref, bp_ref, g2_ref, be2_ref, w1_ref, bm1_ref, w2_ref, bm2_ref,
          o_ref, qkv_s, att_s, xwb, awb):
    f32 = jnp.float32
    pid = pl.program_id(0)

    @pl.when(pid < 8)
    def _phase_qkv():
        for wl in range(2):
            for wc in range(4):
                for i in range(WS):
                    dst = (wl * 4 + wc) * NP + i * WS
                    src = wl * 196 + i * 28 + wc * WS
                    xwb[dst:dst + WS, :] = x_ref[src:src + WS, :]
                xwb[(wl * 4 + wc) * NP + N:(wl * 4 + wc) * NP + NP, :] = (
                    jnp.zeros((NP - N, DIM), f32))
        xv = xwb[...]
        mu = jnp.mean(xv, axis=1, keepdims=True)
        var = jnp.mean((xv - mu) ** 2, axis=1, keepdims=True)
        ln = (xv - mu) * jax.lax.rsqrt(var + 1e-5) * g1_ref[...] + b1_ref[...]

        t_b = pid // 2
        p_b = pid % 2

        @pl.when(pid < 2)
        def _full_qkv():
            res = jax.lax.dot_general(
                ln, wqkv_ref[...], (((1,), (1,)), ((), ())),
                preferred_element_type=f32) + bqkv_ref[...]
            for wloc in range(8):
                a_w = 2 * p_b + wloc // 4
                wc = wloc % 4
                qkv_s[0, pl.ds(((wc * 4 + t_b) * 4 + a_w) * NP, NP), :] = (
                    res[wloc * NP:(wloc + 1) * NP, :DIM] * SCALE)
            qkv_s[1, pl.ds(pid * 448, 448), :] = res[:, DIM:2 * DIM]
            qkv_s[2, pl.ds(pid * 448, 448), :] = res[:, 2 * DIM:]

        @pl.when(pid >= 2)
        def _q_only():
            resq = jax.lax.dot_general(
                ln, wqkv_ref[0:DIM, :], (((1,), (1,)), ((), ())),
                preferred_element_type=f32) + bqkv_ref[:, :DIM]
            for wloc in range(8):
                a_w = 2 * p_b + wloc // 4
                wc = wloc % 4
                qkv_s[0, pl.ds(((wc * 4 + t_b) * 4 + a_w) * NP, NP), :] = (
                    resq[wloc * NP:(wloc + 1) * NP, :] * SCALE)

    @pl.when((pid >= 8) & (pid < 12))
    def _phase_attn():
        r = pid - 8
        qf = qkv_s[0, pl.ds(r * 16 * NP, 16 * NP), :]
        lanes2 = jax.lax.broadcasted_iota(jnp.int32, (4 * NP, 2 * NP), 1)
        laneso = jax.lax.broadcasted_iota(jnp.int32, (4 * NP, 2 * HD), 1)
        z56 = jnp.zeros((NP, HD), f32)
        o1 = jnp.ones((NP, 1), f32)
        zz1 = jnp.zeros((NP, 1), f32)
        sumcols = jnp.concatenate([
            jnp.concatenate([o1, zz1], axis=1),
            jnp.concatenate([zz1, o1], axis=1)], axis=0)
        for b in range(4):
            kf = qkv_s[1, pl.ds((4 * r + b) * NP, NP), :]
            vf = qkv_s[2, pl.ds((4 * r + b) * NP, NP), :]
            for i in range(HEADS // 2):
                sl = slice(i * 2 * HD, (i + 1) * 2 * HD)
                k1 = kf[:, i * 2 * HD:i * 2 * HD + HD]
                k2 = kf[:, i * 2 * HD + HD:(i + 1) * 2 * HD]
                bd_k = jnp.concatenate([
                    jnp.concatenate([k1, z56], axis=1),
                    jnp.concatenate([z56, k2], axis=1)], axis=0)
                s = jax.lax.dot_general(
                    qf[:, sl], bd_k, (((1,), (1,)), ((), ())),
                    preferred_element_type=f32)
                s = (s.reshape(16, NP, 2 * NP) + bias_ref[i][None]
                     ).reshape(16 * NP, 2 * NP)
                v1 = vf[:, i * 2 * HD:i * 2 * HD + HD]
                v2 = vf[:, i * 2 * HD + HD:(i + 1) * 2 * HD]
                bd_v = jnp.concatenate([
                    jnp.concatenate([v1, z56, sumcols[:NP]], axis=1),
                    jnp.concatenate([z56, v2, sumcols[NP:]], axis=1)],
                    axis=0)
                o4 = jnp.zeros((4 * NP, 2 * HD), f32)
                for t in range(4):
                    st = s[t * 4 * NP:(t + 1) * 4 * NP]
                    m1 = jnp.max(st[:, :NP], axis=-1, keepdims=True)
                    m2 = jnp.max(st[:, NP:], axis=-1, keepdims=True)
                    e = jnp.exp(st - jnp.where(lanes2 < NP, m1, m2))
                    oa = jax.lax.dot_general(
                        e, bd_v, (((1,), (0,)), ((), ())),
                        preferred_element_type=f32)
                    rs1 = 1.0 / oa[:, 2 * HD:2 * HD + 1]
                    rs2 = 1.0 / oa[:, 2 * HD + 1:2 * HD + 2]
                    o4 = o4 + oa[:, :2 * HD] * jnp.where(laneso < HD, rs1, rs2)
                o4 = (o4 * 0.25).reshape(4, NP, 2 * HD)
                for a in range(4):
                    att_s[pl.ds((a * 16 + 4 * r + b) * NP, NP), sl] = o4[a]

    @pl.when(pid >= 12)
    def _phase_mlp():
        j = pid - 12
        xwb[...] = att_s[pl.ds(j * 448, 448), :]
        for wl in range(2):
            for wc in range(4):
                for i in range(WS):
                    src = (wl * 4 + wc) * NP + i * WS
                    dst = wl * 196 + i * 28 + wc * WS
                    awb[dst:dst + WS, :] = xwb[src:src + WS, :]
        z = jax.lax.dot_general(
            awb[...], wp_ref[...], (((1,), (1,)), ((), ())),
            preferred_element_type=f32) + bp_ref[...]
        x2 = x_ref[...] + z
        mu = jnp.mean(x2, axis=1, keepdims=True)
        var = jnp.mean((x2 - mu) ** 2, axis=1, keepdims=True)
        ln = (x2 - mu) * jax.lax.rsqrt(var + 1e-5) * g2_ref[...] + be2_ref[...]
        h1 = jax.lax.dot_general(
            ln, w1_ref[...], (((1,), (1,)), ((), ())),
            preferred_element_type=f32) + bm1_ref[...]
        h1 = 0.5 * h1 * (1.0 + jax.lax.erf(h1 * (2.0 ** -0.5)))
        y = jax.lax.dot_general(
            h1, w2_ref[...], (((1,), (1,)), ((), ())),
            preferred_element_type=f32) + bm2_ref[...]
        o_ref[...] = x2 + y


def kernel(x, n1g, n1b, Wqkv, bqkv, rpb, Wproj, bproj, n2g, n2b, W1, b1, W2, b2):
    f32 = jnp.float32
    xf = x.reshape(TOK, DIM)

    rpbg = rpb[_RPI_FLAT].reshape(N, N, HEADS)
    full = jnp.full((HEADS, NP, NP), -1e30, f32).at[:, :N, :N].set(
        rpbg.transpose(2, 0, 1))
    bias2 = jnp.concatenate([full[0::2], full[1::2]], axis=2)

    c0 = lambda i: (0, 0)
    out = pl.pallas_call(
        _body,
        grid=(20,),
        in_specs=[
            pl.BlockSpec((392, DIM),
                         lambda g: (jnp.where(g < 8, g,
                                              jnp.clip(g - 12, 0, 7)), 0)),
            pl.BlockSpec((1, DIM), c0),
            pl.BlockSpec((1, DIM), c0),
            pl.BlockSpec((3 * DIM, DIM), c0),
            pl.BlockSpec((1, 3 * DIM), c0),
            pl.BlockSpec((HEADS // 2, NP, 2 * NP), lambda g: (0, 0, 0)),
            pl.BlockSpec((DIM, DIM), c0),
            pl.BlockSpec((1, DIM), c0),
            pl.BlockSpec((1, DIM), c0),
            pl.BlockSpec((1, DIM), c0),
            pl.BlockSpec((HIDDEN, DIM), c0),
            pl.BlockSpec((1, HIDDEN), c0),
            pl.BlockSpec((DIM, HIDDEN), c0),
            pl.BlockSpec((1, DIM), c0),
        ],
        out_specs=pl.BlockSpec((392, DIM),
                               lambda g: (jnp.where(g >= 12, g - 12, 0), 0)),
        out_shape=jax.ShapeDtypeStruct((TOK, DIM), f32),
        scratch_shapes=[
            pltpu.VMEM((3, TOKP, DIM), f32),
            pltpu.VMEM((TOKP, DIM), f32),
            pltpu.VMEM((448, DIM), f32),
            pltpu.VMEM((392, DIM), f32),
        ],
    )(xf, n1g.reshape(1, DIM), n1b.reshape(1, DIM), Wqkv,
      bqkv.reshape(1, 3 * DIM), bias2, Wproj, bproj.reshape(1, DIM),
      n2g.reshape(1, DIM), n2b.reshape(1, DIM),
      W1, b1.reshape(1, HIDDEN), W2, b2.reshape(1, DIM))

    return out.reshape(B, H * W, DIM)
